# Initial kernel scaffold; baseline (speedup 1.0000x reference)
import jax
import jax.numpy as jnp
from jax.experimental import pallas as pl


def kernel(node_features, edge_index, edge_attr, edge_params, node_params):
    raise NotImplementedError("write your pallas kernel here")


# speedup vs baseline: 2.9279x; 2.9279x over previous
import functools

import jax
import jax.numpy as jnp
from jax import lax
from jax.experimental import pallas as pl
from jax.experimental.pallas import tpu as pltpu
from jax.experimental.pallas import tpu_sc as plsc

H = 128
N = 10000
E = 320000

NC = 2
NS = 16
NW = NC * NS
EPW = E // NW
CHUNK = 80
NCHUNK = EPW // CHUNK
NP = 10240
ROWS_PER_TILE = NP // NS

_f32 = jnp.float32


def _precompute_body(x, w1s, w1r, b1, ps, pr):
    xv = x[...]
    ps[...] = jnp.dot(xv, w1s[...], preferred_element_type=_f32) + b1[...]
    pr[...] = jnp.dot(xv, w1r[...], preferred_element_type=_f32)


def _edge_mlp_body(gs, gr, ea, w1e, w2, b2, w3, b3, w4, b4, g, beta,
                   ue, oe):
    eav = ea[...]
    h = gs[...] + gr[...] + jnp.dot(eav, w1e[...], preferred_element_type=_f32)
    h = jnp.maximum(h, 0.0)
    h = jnp.maximum(jnp.dot(h, w2[...], preferred_element_type=_f32) + b2[...], 0.0)
    h = jnp.maximum(jnp.dot(h, w3[...], preferred_element_type=_f32) + b3[...], 0.0)
    h = jnp.dot(h, w4[...], preferred_element_type=_f32) + b4[...]
    mu = jnp.mean(h, axis=1, keepdims=True)
    d = h - mu
    var = jnp.mean(d * d, axis=1, keepdims=True)
    u = d * lax.rsqrt(var + 1e-5) * g[...] + beta[...]
    ue[...] = u
    oe[...] = eav + u


def _node_mlp_body(x, p0, p1, v1a, v1b, c1, v2, c2, v3, c3, v4, c4, gn, bn,
                   out):
    xv = x[...]
    agg = p0[...] + p1[...]
    h = (jnp.dot(xv, v1a[...], preferred_element_type=_f32)
         + jnp.dot(agg, v1b[...], preferred_element_type=_f32) + c1[...])
    h = jnp.maximum(h, 0.0)
    h = jnp.maximum(jnp.dot(h, v2[...], preferred_element_type=_f32) + c2[...], 0.0)
    h = jnp.maximum(jnp.dot(h, v3[...], preferred_element_type=_f32) + c3[...], 0.0)
    h = jnp.dot(h, v4[...], preferred_element_type=_f32) + c4[...]
    mu = jnp.mean(h, axis=1, keepdims=True)
    d = h - mu
    var = jnp.mean(d * d, axis=1, keepdims=True)
    out[...] = xv + d * lax.rsqrt(var + 1e-5) * gn[...] + bn[...]


def _row_spec(block_rows):
    return pl.BlockSpec((block_rows, H), lambda i: (i, 0))


def _const_spec(shape):
    return pl.BlockSpec(shape, lambda i: (0, 0))


@functools.cache
def _sc_kernels():
    mesh = plsc.VectorSubcoreMesh(core_axis_name="c", subcore_axis_name="s",
                                  num_cores=NC, num_subcores=NS)

    @functools.partial(
        pl.kernel,
        out_type=[jax.ShapeDtypeStruct((E, H), _f32),
                  jax.ShapeDtypeStruct((E, H), _f32)],
        mesh=mesh,
        scratch_types=[
            pltpu.VMEM((CHUNK,), jnp.int32),
            pltpu.VMEM((CHUNK, H), _f32),
            pltpu.VMEM((CHUNK,), jnp.int32),
            pltpu.VMEM((CHUNK, H), _f32),
            pltpu.SemaphoreType.DMA,
            pltpu.SemaphoreType.DMA,
        ],
    )
    def sc_gather(ps_hbm, pr_hbm, send_hbm, recv_hbm, gs_hbm, gr_hbm,
                  idx_s, rows_s, idx_r, rows_r, sem_s, sem_r):
        wid = lax.axis_index("s") * NC + lax.axis_index("c")
        base = wid * EPW

        def body(c, carry):
            off = base + c * CHUNK
            pltpu.sync_copy(send_hbm.at[pl.ds(off, CHUNK)], idx_s)
            pltpu.sync_copy(recv_hbm.at[pl.ds(off, CHUNK)], idx_r)
            a = pltpu.async_copy(ps_hbm.at[idx_s], rows_s, sem_s)
            b = pltpu.async_copy(pr_hbm.at[idx_r], rows_r, sem_r)
            a.wait()
            b.wait()
            pltpu.sync_copy(rows_s, gs_hbm.at[pl.ds(off, CHUNK)])
            pltpu.sync_copy(rows_r, gr_hbm.at[pl.ds(off, CHUNK)])
            return carry

        lax.fori_loop(0, NCHUNK, body, 0)

    @functools.partial(
        pl.kernel,
        out_type=[jax.ShapeDtypeStruct((NP, H), _f32),
                  jax.ShapeDtypeStruct((NP, H), _f32)],
        mesh=mesh,
        scratch_types=[
            pltpu.VMEM((CHUNK,), jnp.int32),
            pltpu.VMEM((CHUNK, H), _f32),
            pltpu.VMEM_SHARED((NP, H), _f32),
        ],
    )
    def sc_scatter(ue_hbm, recv_hbm, zeros_hbm, p0_hbm, p1_hbm,
                   idx_v, rows_v, acc):
        cid = lax.axis_index("c")
        sid = lax.axis_index("s")
        row0 = sid * ROWS_PER_TILE
        pltpu.sync_copy(zeros_hbm.at[pl.ds(row0, ROWS_PER_TILE)],
                        acc.at[pl.ds(row0, ROWS_PER_TILE)])
        plsc.subcore_barrier()

        base = cid * (E // NC) + sid * EPW

        def body(c, carry):
            off = base + c * CHUNK
            pltpu.sync_copy(recv_hbm.at[pl.ds(off, CHUNK)], idx_v)
            pltpu.sync_copy(ue_hbm.at[pl.ds(off, CHUNK)], rows_v)
            pltpu.sync_copy(rows_v, acc.at[idx_v], add=True)
            return carry

        lax.fori_loop(0, NCHUNK, body, 0)
        plsc.subcore_barrier()

        @pl.when(cid == 0)
        def _():
            pltpu.sync_copy(acc.at[pl.ds(row0, ROWS_PER_TILE)],
                            p0_hbm.at[pl.ds(row0, ROWS_PER_TILE)])

        @pl.when(cid == 1)
        def _():
            pltpu.sync_copy(acc.at[pl.ds(row0, ROWS_PER_TILE)],
                            p1_hbm.at[pl.ds(row0, ROWS_PER_TILE)])

    return sc_gather, sc_scatter


def kernel(node_features, edge_index, edge_attr, edge_params, node_params):
    (w1, b1), (w2, b2), (w3, b3), (w4, b4), g, beta = edge_params
    (v1, c1), (v2, c2), (v3, c3), (v4, c4), gn, bn = node_params

    send = edge_index[0].astype(jnp.int32)
    recv = edge_index[1].astype(jnp.int32)

    w1s, w1r, w1e = w1[:H], w1[H:2 * H], w1[2 * H:]
    v1a, v1b = v1[:H], v1[H:]
    row = lambda v: v.reshape(1, H)

    bn_rows = 1000
    ps, pr = pl.pallas_call(
        _precompute_body,
        grid=(N // bn_rows,),
        in_specs=[_row_spec(bn_rows), _const_spec((H, H)), _const_spec((H, H)),
                  _const_spec((1, H))],
        out_specs=[_row_spec(bn_rows), _row_spec(bn_rows)],
        out_shape=[jax.ShapeDtypeStruct((N, H), _f32),
                   jax.ShapeDtypeStruct((N, H), _f32)],
    )(node_features, w1s, w1r, row(b1))

    sc_gather, sc_scatter = _sc_kernels()
    gs, gr = sc_gather(ps, pr, send, recv)

    be_rows = 2000
    ue, out_edges = pl.pallas_call(
        _edge_mlp_body,
        grid=(E // be_rows,),
        in_specs=[_row_spec(be_rows), _row_spec(be_rows), _row_spec(be_rows),
                  _const_spec((H, H)),
                  _const_spec((H, H)), _const_spec((1, H)),
                  _const_spec((H, H)), _const_spec((1, H)),
                  _const_spec((H, H)), _const_spec((1, H)),
                  _const_spec((1, H)), _const_spec((1, H))],
        out_specs=[_row_spec(be_rows), _row_spec(be_rows)],
        out_shape=[jax.ShapeDtypeStruct((E, H), _f32),
                   jax.ShapeDtypeStruct((E, H), _f32)],
    )(gs, gr, edge_attr, w1e, w2, row(b2), w3, row(b3), w4, row(b4),
      row(g), row(beta))

    zeros = jnp.zeros((NP, H), _f32)
    p0, p1 = sc_scatter(ue, recv, zeros)

    out_nodes = pl.pallas_call(
        _node_mlp_body,
        grid=(N // bn_rows,),
        in_specs=[_row_spec(bn_rows), _row_spec(bn_rows), _row_spec(bn_rows),
                  _const_spec((H, H)), _const_spec((H, H)), _const_spec((1, H)),
                  _const_spec((H, H)), _const_spec((1, H)),
                  _const_spec((H, H)), _const_spec((1, H)),
                  _const_spec((H, H)), _const_spec((1, H)),
                  _const_spec((1, H)), _const_spec((1, H))],
        out_specs=_row_spec(bn_rows),
        out_shape=jax.ShapeDtypeStruct((N, H), _f32),
    )(node_features, p0, p1, v1a, v1b, row(c1), v2, row(c2), v3, row(c3),
      v4, row(c4), row(gn), row(bn))

    return (out_nodes, edge_index, out_edges)
